# baseline (speedup 1.0000x reference)
import functools

import jax
import jax.numpy as jnp
from jax import lax
from jax.experimental import pallas as pl
from jax.experimental.pallas import tpu as pltpu
from jax.experimental.pallas import tpu_sc as plsc

B, N, D = 32, 576, 768
ROWS = B * N
NW = 32
RPW = ROWS // NW
CH = 48
NCH = RPW // CH
NL = D // 16


def _sc_body(tok_hbm, embed_hbm, idx_hbm, out_hbm,
             idx_v, rows_v, buf, sem_in0, sem_in1, sem_out0, sem_out1):
    wid = lax.axis_index("s") * 2 + lax.axis_index("c")
    base = wid * RPW

    pltpu.sync_copy(idx_hbm, idx_v)
    pltpu.async_copy(embed_hbm.at[idx_v], rows_v, sem_in0).wait()
    row = [rows_v[0, pl.ds(j * 16, 16)] for j in range(NL)]

    sem_in = (sem_in0, sem_in1)
    sem_out = (sem_out0, sem_out1)

    def start_in(k):
        return pltpu.async_copy(
            tok_hbm.at[pl.ds(base + k * CH, CH)], buf.at[k % 2], sem_in[k % 2])

    def start_out(k):
        return pltpu.async_copy(
            buf.at[k % 2], out_hbm.at[pl.ds(base + k * CH, CH)], sem_out[k % 2])

    in_h = [None] * NCH
    out_h = [None] * NCH
    in_h[0] = start_in(0)
    for k in range(NCH):
        b = k % 2
        if k + 1 < NCH:
            if k >= 1:
                out_h[k - 1].wait()
            in_h[k + 1] = start_in(k + 1)
        in_h[k].wait()

        def row_body(r, carry):
            for j in range(NL):
                sl = pl.ds(j * 16, 16)
                buf[b, r, sl] = buf[b, r, sl] + row[j]
            return carry
        lax.fori_loop(0, CH, row_body, None)
        out_h[k] = start_out(k)
    out_h[NCH - 2].wait()
    out_h[NCH - 1].wait()


def kernel(tokens, embed, frame_idx):
    idx8 = jnp.full((8,), frame_idx, dtype=jnp.int32)
    tok2 = tokens.reshape(ROWS, D)
    sc = functools.partial(
        pl.kernel,
        mesh=plsc.VectorSubcoreMesh(core_axis_name="c", subcore_axis_name="s"),
        out_type=jax.ShapeDtypeStruct((ROWS, D), jnp.float32),
        scratch_types=[
            pltpu.VMEM((8,), jnp.int32),
            pltpu.VMEM((8, D), jnp.float32),
            pltpu.VMEM((2, CH, D), jnp.float32),
            pltpu.SemaphoreType.DMA,
            pltpu.SemaphoreType.DMA,
            pltpu.SemaphoreType.DMA,
            pltpu.SemaphoreType.DMA,
        ],
    )(_sc_body)
    out = sc(tok2, embed, idx8)
    return out.reshape(B, N, D)


# speedup vs baseline: 2.0865x; 2.0865x over previous
import jax
import jax.numpy as jnp
from jax.experimental import pallas as pl
from jax.experimental.pallas import tpu as pltpu

B, N, D = 32, 576, 768
ROWS = B * N
NCHUNK = 3
CHUNK = ROWS // NCHUNK


def _body(idx_ref, embed_ref, tok_hbm, out_hbm, buf, in_sem, out_sem):
    row = embed_ref[pl.ds(idx_ref[0], 1), :]
    loads = [
        pltpu.make_async_copy(
            tok_hbm.at[pl.ds(k * CHUNK, CHUNK)], buf.at[k], in_sem.at[k])
        for k in range(NCHUNK)
    ]
    stores = [
        pltpu.make_async_copy(
            buf.at[k], out_hbm.at[pl.ds(k * CHUNK, CHUNK)], out_sem.at[k])
        for k in range(NCHUNK)
    ]
    for ld in loads:
        ld.start()
    for k in range(NCHUNK):
        loads[k].wait()
        buf[k] = buf[k] + row
        stores[k].start()
    for st in stores:
        st.wait()


def kernel(tokens, embed, frame_idx):
    idx = jnp.asarray(frame_idx, dtype=jnp.int32).reshape((1,))
    tok2 = tokens.reshape(ROWS, D)
    out = pl.pallas_call(
        _body,
        in_specs=[
            pl.BlockSpec(memory_space=pltpu.MemorySpace.SMEM),
            pl.BlockSpec(memory_space=pltpu.MemorySpace.VMEM),
            pl.BlockSpec(memory_space=pltpu.MemorySpace.HBM),
        ],
        out_specs=pl.BlockSpec(memory_space=pltpu.MemorySpace.HBM),
        out_shape=jax.ShapeDtypeStruct((ROWS, D), tokens.dtype),
        scratch_shapes=[
            pltpu.VMEM((NCHUNK, CHUNK, D), jnp.float32),
            pltpu.SemaphoreType.DMA((NCHUNK,)),
            pltpu.SemaphoreType.DMA((NCHUNK,)),
        ],
        compiler_params=pltpu.CompilerParams(
            vmem_limit_bytes=60 * 1024 * 1024,
        ),
    )(idx, embed, tok2)
    return out.reshape(B, N, D)
